# baseline (speedup 1.0000x reference)
import functools

import jax
import jax.numpy as jnp
from jax import lax
from jax.experimental import pallas as pl
from jax.experimental.pallas import tpu as pltpu
from jax.experimental.pallas import tpu_sc as plsc

P0, P1, P2 = 100, 100, 100
Q0, Q1, Q2 = 4, 4, 4
R1, R2 = 32, 32
B = 16384
DT = Q1 * R2
TROW = Q0 * DT
NPAIR = P0 * P1

L = 16
NC = 2
NS = 16
NW = NC * NS
BPW = B // NW
CH = 64
NCH = BPW // CH
NG = CH // L
RC = 8

CI = 20


def _tc_pair_table(g0_ref, g1_ref, out_ref):
    for j in range(CI):
        g1 = g1_ref[j]
        for q0 in range(Q0):
            mm = jnp.dot(g0_ref[:, q0, :], g1, preferred_element_type=jnp.float32)
            out_ref[pl.ds(j * P0, P0), pl.ds(q0 * DT, DT)] = mm


def _sc_lookup_body(idx_hbm, t_hbm, g2t_hbm, out_hbm,
                    idxv, pairv, i2v, trows, crows, outv, sem1, sem2):
    wid = lax.axis_index("s") * NC + lax.axis_index("c")
    base = wid * BPW
    pltpu.sync_copy(idx_hbm.at[pl.ds(base, BPW)], idxv)
    lane = lax.iota(jnp.int32, L)

    def chunk_body(ci, carry):
        off = ci * CH
        for g in range(NG):
            v = idxv[pl.ds(off + g * L, L)]
            i0 = lax.div(v, P1 * P2)
            rem = v - i0 * (P1 * P2)
            i1 = lax.div(rem, P2)
            i2 = rem - i1 * P2
            pairv[pl.ds(g * L, L)] = i1 * P0 + i0
            i2v[pl.ds(g * L, L)] = i2
        cp1 = pltpu.async_copy(t_hbm.at[pairv], trows, sem1)
        cp2 = pltpu.async_copy(g2t_hbm.at[i2v], crows, sem2)
        cp1.wait()
        cp2.wait()

        @plsc.parallel_loop(0, NG, 1, unroll=1)
        def group_body(g):
            rows = lane + g * L
            for rc in range(R2 // RC):
                cregs = [[plsc.load_gather(crows, [rows, jnp.full((L,), q2 * R2 + rc * RC + r, jnp.int32)])
                          for r in range(RC)] for q2 in range(Q2)]
                for q0 in range(Q0):
                    for q1 in range(Q1):
                        tbase = q0 * DT + q1 * R2 + rc * RC
                        tregs = [plsc.load_gather(trows, [rows, jnp.full((L,), tbase + r, jnp.int32)])
                                 for r in range(RC)]
                        for q2 in range(Q2):
                            p = [tregs[r] * cregs[q2][r] for r in range(RC)]
                            while len(p) > 1:
                                p = [p[i] + p[i + 1] for i in range(0, len(p) - 1, 2)] \
                                    + ([p[-1]] if len(p) % 2 else [])
                            s = p[0]
                            ocol = jnp.full((L,), q0 * 16 + q1 * 4 + q2, jnp.int32)
                            if rc == 0:
                                plsc.store_scatter(outv, [rows, ocol], s)
                            else:
                                plsc.addupdate_scatter(outv, [rows, ocol], s)
        pltpu.sync_copy(outv, out_hbm.at[pl.ds(base + off, CH)])
        return carry

    lax.fori_loop(0, NCH, chunk_body, 0)


def kernel(indices, G0, G1, G2):
    idx = indices.astype(jnp.int32)
    g0q = G0.reshape(P0, Q0, R1)
    g1r = G1.reshape(P1, R1, DT)
    g2t = G2.reshape(P2, R2, Q2).transpose(0, 2, 1).reshape(P2, Q2 * R2)

    t_table = pl.pallas_call(
        _tc_pair_table,
        grid=(P1 // CI,),
        in_specs=[
            pl.BlockSpec((P0, Q0, R1), lambda i: (0, 0, 0)),
            pl.BlockSpec((CI, R1, DT), lambda i: (i, 0, 0)),
        ],
        out_specs=pl.BlockSpec((CI * P0, TROW), lambda i: (i, 0)),
        out_shape=jax.ShapeDtypeStruct((NPAIR, TROW), jnp.float32),
    )(g0q, g1r)

    sc = functools.partial(
        pl.kernel,
        mesh=plsc.VectorSubcoreMesh(core_axis_name="c", subcore_axis_name="s"),
        out_type=jax.ShapeDtypeStruct((B, Q0 * Q1 * Q2), jnp.float32),
        compiler_params=pltpu.CompilerParams(
            needs_layout_passes=False, disable_bounds_checks=True),
        scratch_types=[
            pltpu.VMEM((BPW,), jnp.int32),
            pltpu.VMEM((CH,), jnp.int32),
            pltpu.VMEM((CH,), jnp.int32),
            pltpu.VMEM((CH, TROW), jnp.float32),
            pltpu.VMEM((CH, Q2 * R2), jnp.float32),
            pltpu.VMEM((CH, Q0 * Q1 * Q2), jnp.float32),
            pltpu.SemaphoreType.DMA,
            pltpu.SemaphoreType.DMA,
        ],
    )(_sc_lookup_body)
    return sc(idx, t_table, g2t)


# speedup vs baseline: 1.2821x; 1.2821x over previous
import functools

import jax
import jax.numpy as jnp
from jax import lax
from jax.experimental import pallas as pl
from jax.experimental.pallas import tpu as pltpu
from jax.experimental.pallas import tpu_sc as plsc

P0, P1, P2 = 100, 100, 100
Q0, Q1, Q2 = 4, 4, 4
R1, R2 = 32, 32
B = 16384
DT = Q1 * R2
TROW = Q0 * DT
NPAIR = P0 * P1

L = 16
NC = 2
NS = 16
NW = NC * NS
BPW = B // NW
CH = 64
NCH = BPW // CH
NG = CH // L
RC = 8

CI = 20


def _tc_pair_table(g0_ref, g1_ref, out_ref):
    for j in range(CI):
        g1 = g1_ref[j]
        for q0 in range(Q0):
            mm = jnp.dot(g0_ref[:, q0, :], g1, preferred_element_type=jnp.float32)
            out_ref[pl.ds(j * P0, P0), pl.ds(q0 * DT, DT)] = mm


def _sc_lookup_body(idx_hbm, t_hbm, g2t_hbm, out_hbm,
                    idxv, pairv, i2v, trows, crows, outv, sem1, sem2):
    wid = lax.axis_index("s") * NC + lax.axis_index("c")
    base = wid * BPW
    pltpu.sync_copy(idx_hbm.at[pl.ds(base, BPW)], idxv)
    lane = lax.iota(jnp.int32, L)

    def chunk_body(ci, carry):
        off = ci * CH
        for g in range(NG):
            v = idxv[pl.ds(off + g * L, L)]
            i0 = lax.div(v, P1 * P2)
            rem = v - i0 * (P1 * P2)
            i1 = lax.div(rem, P2)
            i2 = rem - i1 * P2
            pairv[pl.ds(g * L, L)] = i1 * P0 + i0
            i2v[pl.ds(g * L, L)] = i2
        cp1 = pltpu.async_copy(t_hbm.at[pairv], trows, sem1)
        cp2 = pltpu.async_copy(g2t_hbm.at[i2v], crows, sem2)
        cp1.wait()
        cp2.wait()

        @plsc.parallel_loop(0, NG, 1, unroll=1)
        def group_body(g):
            rows = lane + g * L
            for rc in range(R2 // RC):
                skc = [rc * RC | ((lane + r) & (RC - 1)) for r in range(RC)]
                q2sks = [(lane + q2) & (Q2 - 1) for q2 in range(Q2)]
                cregs = [[plsc.load_gather(crows, [rows, (q2sks[q2] << 5) | skc[r]])
                          for r in range(RC)] for q2 in range(Q2)]
                for jj in range(Q0 * Q1):
                    qsk = (lane + jj) & (Q0 * Q1 - 1)
                    qbase = qsk << 5
                    tregs = [plsc.load_gather(trows, [rows, qbase | skc[r]])
                             for r in range(RC)]
                    for q2 in range(Q2):
                        p = [tregs[r] * cregs[q2][r] for r in range(RC)]
                        while len(p) > 1:
                            p = [p[i] + p[i + 1] for i in range(0, len(p) - 1, 2)] \
                                + ([p[-1]] if len(p) % 2 else [])
                        s = p[0]
                        ocol = (qsk << 2) | q2sks[q2]
                        if rc == 0:
                            plsc.store_scatter(outv, [rows, ocol], s)
                        else:
                            plsc.addupdate_scatter(outv, [rows, ocol], s)
        pltpu.sync_copy(outv, out_hbm.at[pl.ds(base + off, CH)])
        return carry

    lax.fori_loop(0, NCH, chunk_body, 0)


def kernel(indices, G0, G1, G2):
    idx = indices.astype(jnp.int32)
    g0q = G0.reshape(P0, Q0, R1)
    g1r = G1.reshape(P1, R1, DT)
    g2t = G2.reshape(P2, R2, Q2).transpose(0, 2, 1).reshape(P2, Q2 * R2)

    t_table = pl.pallas_call(
        _tc_pair_table,
        grid=(P1 // CI,),
        in_specs=[
            pl.BlockSpec((P0, Q0, R1), lambda i: (0, 0, 0)),
            pl.BlockSpec((CI, R1, DT), lambda i: (i, 0, 0)),
        ],
        out_specs=pl.BlockSpec((CI * P0, TROW), lambda i: (i, 0)),
        out_shape=jax.ShapeDtypeStruct((NPAIR, TROW), jnp.float32),
    )(g0q, g1r)

    sc = functools.partial(
        pl.kernel,
        mesh=plsc.VectorSubcoreMesh(core_axis_name="c", subcore_axis_name="s"),
        out_type=jax.ShapeDtypeStruct((B, Q0 * Q1 * Q2), jnp.float32),
        compiler_params=pltpu.CompilerParams(
            needs_layout_passes=False, disable_bounds_checks=True),
        scratch_types=[
            pltpu.VMEM((BPW,), jnp.int32),
            pltpu.VMEM((CH,), jnp.int32),
            pltpu.VMEM((CH,), jnp.int32),
            pltpu.VMEM((CH, TROW), jnp.float32),
            pltpu.VMEM((CH, Q2 * R2), jnp.float32),
            pltpu.VMEM((CH, Q0 * Q1 * Q2), jnp.float32),
            pltpu.SemaphoreType.DMA,
            pltpu.SemaphoreType.DMA,
        ],
    )(_sc_lookup_body)
    return sc(idx, t_table, g2t)


# speedup vs baseline: 1.3515x; 1.0541x over previous
import functools

import jax
import jax.numpy as jnp
from jax import lax
from jax.experimental import pallas as pl
from jax.experimental.pallas import tpu as pltpu
from jax.experimental.pallas import tpu_sc as plsc

P0, P1, P2 = 100, 100, 100
Q0, Q1, Q2 = 4, 4, 4
R1, R2 = 32, 32
B = 16384
DT = Q1 * R2
TROW = Q0 * DT
NPAIR = P0 * P1

L = 16
NC = 2
NS = 16
NW = NC * NS
BPW = B // NW
CH = 64
NCH = BPW // CH
NG = CH // L
RC = 8

CI = 20


def _tc_pair_table(g0_ref, g1_ref, out_ref):
    for j in range(CI):
        g1 = g1_ref[j]
        for q0 in range(Q0):
            mm = jnp.dot(g0_ref[:, q0, :], g1, preferred_element_type=jnp.float32)
            out_ref[pl.ds(j * P0, P0), pl.ds(q0 * DT, DT)] = mm


def _sc_lookup_body(idx_hbm, t_hbm, g2t_hbm, out_hbm,
                    idxv, pairv, i2v, trows, crows, outv,
                    sem1a, sem1b, sem2a, sem2b, sem3a, sem3b):
    wid = lax.axis_index("s") * NC + lax.axis_index("c")
    base = wid * BPW
    pltpu.sync_copy(idx_hbm.at[pl.ds(base, BPW)], idxv)
    lane = lax.iota(jnp.int32, L)

    def gather_cps(slot):
        sl = pl.ds(slot * CH, CH)
        ts, cs = (sem1a, sem2a) if slot == 0 else (sem1b, sem2b)
        return (pltpu.make_async_copy(t_hbm.at[pairv.at[sl]], trows.at[sl], ts),
                pltpu.make_async_copy(g2t_hbm.at[i2v.at[sl]], crows.at[sl], cs))

    def decompose(ci, slot_base):
        off = ci * CH
        for g in range(NG):
            v = idxv[pl.ds(off + g * L, L)]
            i0 = lax.div(v, P1 * P2)
            rem = v - i0 * (P1 * P2)
            i1 = lax.div(rem, P2)
            i2 = rem - i1 * P2
            pairv[pl.ds(slot_base + g * L, L)] = i1 * P0 + i0
            i2v[pl.ds(slot_base + g * L, L)] = i2

    def fire(ci, slot):
        decompose(ci, slot * CH)
        cp1, cp2 = gather_cps(slot)
        cp1.start()
        cp2.start()

    fire(0, 0)

    def chunk_body(ci, carry):
        slot = jnp.bitwise_and(ci, 1)

        @pl.when(ci + 1 < NCH)
        def _():
            nci = ci + 1

            @pl.when(slot == 0)
            def _():
                decompose(nci, CH)

            @pl.when(slot == 1)
            def _():
                decompose(nci, 0)

        for s in range(2):
            @pl.when(jnp.logical_and(slot == s, ci + 1 < NCH))
            def _(s=s):
                cp1, cp2 = gather_cps(1 - s)
                cp1.start()
                cp2.start()
            @pl.when(jnp.logical_and(slot == s, ci >= 2))
            def _(s=s):
                osl = pl.ds(s * CH, CH)
                osem = sem3a if s == 0 else sem3b
                pltpu.make_async_copy(
                    outv.at[osl], out_hbm.at[pl.ds(base, CH)], osem).wait()
            @pl.when(slot == s)
            def _(s=s):
                cp1, cp2 = gather_cps(s)
                cp1.wait()
                cp2.wait()

        rbase = slot * CH

        @plsc.parallel_loop(0, NG, 1, unroll=1)
        def group_body(g):
            rows = lane + g * L + rbase
            for rc in range(R2 // RC):
                skc = [rc * RC | ((lane + r) & (RC - 1)) for r in range(RC)]
                q2sks = [(lane + q2) & (Q2 - 1) for q2 in range(Q2)]
                cregs = [[plsc.load_gather(crows, [rows, (q2sks[q2] << 5) | skc[r]])
                          for r in range(RC)] for q2 in range(Q2)]
                for jj in range(Q0 * Q1):
                    qsk = (lane + jj) & (Q0 * Q1 - 1)
                    qbase = qsk << 5
                    tregs = [plsc.load_gather(trows, [rows, qbase | skc[r]])
                             for r in range(RC)]
                    for q2 in range(Q2):
                        p = [tregs[r] * cregs[q2][r] for r in range(RC)]
                        while len(p) > 1:
                            p = [p[i] + p[i + 1] for i in range(0, len(p) - 1, 2)] \
                                + ([p[-1]] if len(p) % 2 else [])
                        s = p[0]
                        ocol = (qsk << 2) | q2sks[q2]
                        if rc == 0:
                            plsc.store_scatter(outv, [rows, ocol], s)
                        else:
                            plsc.addupdate_scatter(outv, [rows, ocol], s)

        for s in range(2):
            @pl.when(slot == s)
            def _(s=s):
                osem = sem3a if s == 0 else sem3b
                pltpu.async_copy(outv.at[pl.ds(s * CH, CH)],
                                 out_hbm.at[pl.ds(base + ci * CH, CH)], osem)
        return carry

    lax.fori_loop(0, NCH, chunk_body, 0)
    pltpu.make_async_copy(outv.at[pl.ds(0, CH)],
                          out_hbm.at[pl.ds(base, CH)], sem3a).wait()
    pltpu.make_async_copy(outv.at[pl.ds(CH, CH)],
                          out_hbm.at[pl.ds(base, CH)], sem3b).wait()


def kernel(indices, G0, G1, G2):
    idx = indices.astype(jnp.int32)
    g0q = G0.reshape(P0, Q0, R1)
    g1r = G1.reshape(P1, R1, DT)
    g2t = G2.reshape(P2, R2, Q2).transpose(0, 2, 1).reshape(P2, Q2 * R2)

    t_table = pl.pallas_call(
        _tc_pair_table,
        grid=(P1 // CI,),
        in_specs=[
            pl.BlockSpec((P0, Q0, R1), lambda i: (0, 0, 0)),
            pl.BlockSpec((CI, R1, DT), lambda i: (i, 0, 0)),
        ],
        out_specs=pl.BlockSpec((CI * P0, TROW), lambda i: (i, 0)),
        out_shape=jax.ShapeDtypeStruct((NPAIR, TROW), jnp.float32),
    )(g0q, g1r)

    sc = functools.partial(
        pl.kernel,
        mesh=plsc.VectorSubcoreMesh(core_axis_name="c", subcore_axis_name="s"),
        out_type=jax.ShapeDtypeStruct((B, Q0 * Q1 * Q2), jnp.float32),
        compiler_params=pltpu.CompilerParams(
            needs_layout_passes=False, disable_bounds_checks=True),
        scratch_types=[
            pltpu.VMEM((BPW,), jnp.int32),
            pltpu.VMEM((2 * CH,), jnp.int32),
            pltpu.VMEM((2 * CH,), jnp.int32),
            pltpu.VMEM((2 * CH, TROW), jnp.float32),
            pltpu.VMEM((2 * CH, Q2 * R2), jnp.float32),
            pltpu.VMEM((2 * CH, Q0 * Q1 * Q2), jnp.float32),
            pltpu.SemaphoreType.DMA,
            pltpu.SemaphoreType.DMA,
            pltpu.SemaphoreType.DMA,
            pltpu.SemaphoreType.DMA,
            pltpu.SemaphoreType.DMA,
            pltpu.SemaphoreType.DMA,
        ],
    )(_sc_lookup_body)
    return sc(idx, t_table, g2t)


# speedup vs baseline: 1.5884x; 1.1753x over previous
import functools

import jax
import jax.numpy as jnp
from jax import lax
from jax.experimental import pallas as pl
from jax.experimental.pallas import tpu as pltpu
from jax.experimental.pallas import tpu_sc as plsc

P0, P1, P2 = 100, 100, 100
Q0, Q1, Q2 = 4, 4, 4
R1, R2 = 32, 32
B = 16384
DT = Q1 * R2
TROW = Q0 * DT
NPAIR = P0 * P1

L = 16
NC = 2
NS = 16
NW = NC * NS
BPW = B // NW
CH = 64
NCH = BPW // CH
NG = CH // L
RC = 8

CI = 20


def _tc_pair_table(g0_ref, g1_ref, out_ref):
    for j in range(CI):
        g1 = g1_ref[j]
        for q0 in range(Q0):
            mm = jnp.dot(g0_ref[:, q0, :], g1, preferred_element_type=jnp.float32)
            out_ref[pl.ds(j * P0, P0), pl.ds(q0 * DT, DT)] = mm


def _sc_lookup_body(idx_hbm, t_hbm, g2t_hbm, out_hbm,
                    idxv, pairv, i2v, trows, crows, outv,
                    sem1a, sem1b, sem2a, sem2b, sem3a, sem3b):
    wid = lax.axis_index("s") * NC + lax.axis_index("c")
    base = wid * BPW
    pltpu.sync_copy(idx_hbm.at[pl.ds(base, BPW)], idxv)
    lane = lax.iota(jnp.int32, L)

    def gather_cps(slot):
        sl = pl.ds(slot * CH, CH)
        ts, cs = (sem1a, sem2a) if slot == 0 else (sem1b, sem2b)
        return (pltpu.make_async_copy(t_hbm.at[pairv.at[sl]], trows.at[sl], ts),
                pltpu.make_async_copy(g2t_hbm.at[i2v.at[sl]], crows.at[sl], cs))

    def decompose(ci, slot_base):
        off = ci * CH
        for g in range(NG):
            v = idxv[pl.ds(off + g * L, L)]
            i0 = lax.div(v, P1 * P2)
            rem = v - i0 * (P1 * P2)
            i1 = lax.div(rem, P2)
            i2 = rem - i1 * P2
            pairv[pl.ds(slot_base + g * L, L)] = i1 * P0 + i0
            i2v[pl.ds(slot_base + g * L, L)] = i2

    def fire(ci, slot):
        decompose(ci, slot * CH)
        cp1, cp2 = gather_cps(slot)
        cp1.start()
        cp2.start()

    fire(0, 0)

    def chunk_body(ci, carry):
        slot = jnp.bitwise_and(ci, 1)

        @pl.when(ci + 1 < NCH)
        def _():
            nci = ci + 1

            @pl.when(slot == 0)
            def _():
                decompose(nci, CH)

            @pl.when(slot == 1)
            def _():
                decompose(nci, 0)

        for s in range(2):
            @pl.when(jnp.logical_and(slot == s, ci + 1 < NCH))
            def _(s=s):
                cp1, cp2 = gather_cps(1 - s)
                cp1.start()
                cp2.start()
            @pl.when(jnp.logical_and(slot == s, ci >= 2))
            def _(s=s):
                osl = pl.ds(s * CH, CH)
                osem = sem3a if s == 0 else sem3b
                pltpu.make_async_copy(
                    outv.at[osl], out_hbm.at[pl.ds(base, CH)], osem).wait()
            @pl.when(slot == s)
            def _(s=s):
                cp1, cp2 = gather_cps(s)
                cp1.wait()
                cp2.wait()

        rbase = slot * CH

        @plsc.parallel_loop(0, NG, 1, unroll=1)
        def group_body(g):
            rows = lane + g * L + rbase
            NQ = Q0 * Q1
            for rc in range(R2 // RC):
                skc = [rc * RC | ((lane + r) & (RC - 1)) for r in range(RC)]
                q2sks = [(lane + q2) & (Q2 - 1) for q2 in range(Q2)]
                cregs = [[plsc.load_gather(crows, [rows, (q2sks[q2] << 5) | skc[r]])
                          for r in range(RC)] for q2 in range(Q2)]

                def tload(jj):
                    qsk = (lane + jj) & (NQ - 1)
                    return qsk, [plsc.load_gather(trows, [rows, (qsk << 5) | skc[r]])
                                 for r in range(RC)]

                cur = tload(0)
                for jj in range(NQ):
                    nxt = tload(jj + 1) if jj + 1 < NQ else None
                    qsk, tregs = cur
                    for q2 in range(Q2):
                        p = [tregs[r] * cregs[q2][r] for r in range(RC)]
                        while len(p) > 1:
                            p = [p[i] + p[i + 1] for i in range(0, len(p) - 1, 2)] \
                                + ([p[-1]] if len(p) % 2 else [])
                        s = p[0]
                        ocol = (qsk << 2) | q2sks[q2]
                        if rc == 0:
                            plsc.store_scatter(outv, [rows, ocol], s)
                        else:
                            plsc.addupdate_scatter(outv, [rows, ocol], s)
                    cur = nxt

        for s in range(2):
            @pl.when(slot == s)
            def _(s=s):
                osem = sem3a if s == 0 else sem3b
                pltpu.async_copy(outv.at[pl.ds(s * CH, CH)],
                                 out_hbm.at[pl.ds(base + ci * CH, CH)], osem)
        return carry

    lax.fori_loop(0, NCH, chunk_body, 0)
    pltpu.make_async_copy(outv.at[pl.ds(0, CH)],
                          out_hbm.at[pl.ds(base, CH)], sem3a).wait()
    pltpu.make_async_copy(outv.at[pl.ds(CH, CH)],
                          out_hbm.at[pl.ds(base, CH)], sem3b).wait()


def kernel(indices, G0, G1, G2):
    idx = indices.astype(jnp.int32)
    g0q = G0.reshape(P0, Q0, R1)
    g1r = G1.reshape(P1, R1, DT)
    g2t = G2.reshape(P2, R2, Q2).transpose(0, 2, 1).reshape(P2, Q2 * R2)

    t_table = pl.pallas_call(
        _tc_pair_table,
        grid=(P1 // CI,),
        in_specs=[
            pl.BlockSpec((P0, Q0, R1), lambda i: (0, 0, 0)),
            pl.BlockSpec((CI, R1, DT), lambda i: (i, 0, 0)),
        ],
        out_specs=pl.BlockSpec((CI * P0, TROW), lambda i: (i, 0)),
        out_shape=jax.ShapeDtypeStruct((NPAIR, TROW), jnp.float32),
    )(g0q, g1r)

    sc = functools.partial(
        pl.kernel,
        mesh=plsc.VectorSubcoreMesh(core_axis_name="c", subcore_axis_name="s"),
        out_type=jax.ShapeDtypeStruct((B, Q0 * Q1 * Q2), jnp.float32),
        compiler_params=pltpu.CompilerParams(
            needs_layout_passes=False, disable_bounds_checks=True),
        scratch_types=[
            pltpu.VMEM((BPW,), jnp.int32),
            pltpu.VMEM((2 * CH,), jnp.int32),
            pltpu.VMEM((2 * CH,), jnp.int32),
            pltpu.VMEM((2 * CH, TROW), jnp.float32),
            pltpu.VMEM((2 * CH, Q2 * R2), jnp.float32),
            pltpu.VMEM((2 * CH, Q0 * Q1 * Q2), jnp.float32),
            pltpu.SemaphoreType.DMA,
            pltpu.SemaphoreType.DMA,
            pltpu.SemaphoreType.DMA,
            pltpu.SemaphoreType.DMA,
            pltpu.SemaphoreType.DMA,
            pltpu.SemaphoreType.DMA,
        ],
    )(_sc_lookup_body)
    return sc(idx, t_table, g2t)
